# baseline (speedup 1.0000x reference)
import functools
import math

import jax
import jax.numpy as jnp
from jax import lax
from jax.experimental import pallas as pl
from jax.experimental.pallas import tpu as pltpu
from jax.experimental.pallas import tpu_sc as plsc

_L = 64
_K = 128
_B = 4096
_NW = 32
_BTC = 2048
_BSC = _B - _BTC
_LG = 2
_BG = 16
_LW = _L // _LG
_BW = _BSC // _BG
_NC = _BW // 16
_KC = _K // 16
_C1 = 0.5 * math.log(2.0 * math.pi)
_LN2 = math.log(2.0)
_NEG_BIG = -3.0e38
_S_MIN = 1e-25

_W = 2 * _L
_RTC = _BTC // 2
_BB = 64


_GATHER_DNUMS = lax.GatherDimensionNumbers(
    offset_dims=(), collapsed_slice_dims=(0,), start_index_map=(0,)
)


def _shuf(v, idx):
    return lax.gather(
        v, idx.reshape(16, 1), _GATHER_DNUMS, (1,),
        mode=lax.GatherScatterMode.PROMISE_IN_BOUNDS,
    )


def _butterfly(v, op):
    lane = lax.iota(jnp.int32, 16)
    for b in range(4):
        v = op(v, _shuf(v, lane ^ (1 << b)))
    return v


def _log16(v):
    bits = lax.bitcast_convert_type(v, jnp.int32)
    e = (bits >> 23) - 127
    mbits = (bits & jnp.int32(0x007FFFFF)) | jnp.int32(0x3F800000)
    m = lax.bitcast_convert_type(mbits, jnp.float32)
    big = m > 1.4142135
    m = jnp.where(big, 0.5 * m, m)
    ef = e.astype(jnp.float32) + jnp.where(big, 1.0, 0.0)
    t = (m - 1.0) / (m + 1.0)
    u = t * t
    p = 2.0 * t * (1.0 + u * (1.0 / 3.0 + u * (0.2 + u * (1.0 / 7.0))))
    return ef * _LN2 + p


def _sc_body(zt, mut, lvt, wvec, out3, zv, muv, lvv, outv, av, cv, dv, wv):
    cid = lax.axis_index("c")
    sid = lax.axis_index("s")
    wid = cid * 16 + sid
    lg = wid >> 3
    pltpu.sync_copy(zt.at[wid], zv)
    pltpu.sync_copy(mut.at[pl.ds(lg * _LW, _LW)], muv)
    pltpu.sync_copy(lvt.at[pl.ds(lg * _LW, _LW)], lvv)
    pltpu.sync_copy(wvec, wv)

    wmax_acc = wv[pl.ds(0, 16)]
    for kc in range(1, _KC):
        wmax_acc = jnp.maximum(wmax_acc, wv[pl.ds(kc * 16, 16)])
    wm = _butterfly(wmax_acc, jnp.maximum)
    se = jnp.zeros((16,), jnp.float32)
    for kc in range(_KC):
        se = se + jnp.exp(wv[pl.ds(kc * 16, 16)] - wm)
    lse16 = wm + _log16(_butterfly(se, jnp.add))

    def l_body(l, carry):
        pkmax = jnp.full((16,), _NEG_BIG)
        for kc in range(_KC):
            sl = pl.ds(kc * 16, 16)
            lv16 = lvv[l, sl]
            mu16 = muv[l, sl]
            prec = jnp.exp(-lv16)
            a16 = -0.5 * prec
            c16 = prec * mu16
            pk16 = (-_C1 - 0.5 * lv16) + (wv[sl] - lse16)
            av[sl] = a16
            cv[sl] = c16
            dv[sl] = pk16 + a16 * (mu16 * mu16)
            pkmax = jnp.maximum(pkmax, pk16)
        m0 = _butterfly(pkmax, jnp.maximum)
        for kc in range(_KC):
            sl = pl.ds(kc * 16, 16)
            dv[sl] = dv[sl] - m0

        z8 = [zv[l, pl.ds(ci * 16, 16)] for ci in range(_NC)]
        z28 = [zz * zz for zz in z8]

        def k_body(kc, ss):
            a16 = av[pl.ds(kc * 16, 16)]
            c16 = cv[pl.ds(kc * 16, 16)]
            d16 = dv[pl.ds(kc * 16, 16)]
            for j in range(16):
                ak = a16[j]
                ck = c16[j]
                dk = d16[j]
                ss = tuple(
                    ss[ci] + jnp.exp((ak * z28[ci] + ck * z8[ci]) + dk)
                    for ci in range(_NC)
                )
            return ss

        s8 = lax.fori_loop(
            0, _KC, k_body,
            tuple(jnp.zeros((16,), jnp.float32) for _ in range(_NC)),
        )

        smin = s8[0]
        for ci in range(1, _NC):
            smin = jnp.minimum(smin, s8[ci])
        bad = _butterfly(smin, jnp.minimum)[0] < _S_MIN

        for ci in range(_NC):
            outv[l, pl.ds(ci * 16, 16)] = _log16(s8[ci]) + m0

        @pl.when(bad)
        def redo():
            def mx_body(kc, mm):
                a16 = av[pl.ds(kc * 16, 16)]
                c16 = cv[pl.ds(kc * 16, 16)]
                d16 = dv[pl.ds(kc * 16, 16)]
                for j in range(16):
                    ak = a16[j]
                    ck = c16[j]
                    dk = d16[j]
                    mm = tuple(
                        jnp.maximum(mm[ci], (ak * z28[ci] + ck * z8[ci]) + dk)
                        for ci in range(_NC)
                    )
                return mm

            m8 = lax.fori_loop(
                0, _KC, mx_body,
                tuple(jnp.full((16,), _NEG_BIG) for _ in range(_NC)),
            )

            def s2_body(kc, ss):
                a16 = av[pl.ds(kc * 16, 16)]
                c16 = cv[pl.ds(kc * 16, 16)]
                d16 = dv[pl.ds(kc * 16, 16)]
                for j in range(16):
                    ak = a16[j]
                    ck = c16[j]
                    dk = d16[j]
                    ss = tuple(
                        ss[ci] + jnp.exp(((ak * z28[ci] + ck * z8[ci]) + dk) - m8[ci])
                        for ci in range(_NC)
                    )
                return ss

            s28 = lax.fori_loop(
                0, _KC, s2_body,
                tuple(jnp.zeros((16,), jnp.float32) for _ in range(_NC)),
            )
            for ci in range(_NC):
                outv[l, pl.ds(ci * 16, 16)] = (m8[ci] + _log16(s28[ci])) + m0

        return carry

    lax.fori_loop(0, _LW, l_body, 0)
    pltpu.sync_copy(outv, out3.at[wid])


_sc_kernel = functools.partial(
    pl.kernel,
    mesh=plsc.VectorSubcoreMesh(core_axis_name="c", subcore_axis_name="s"),
    out_type=jax.ShapeDtypeStruct((_NW, _LW, _BW), jnp.float32),
    scratch_types=[
        pltpu.VMEM((_LW, _BW), jnp.float32),
        pltpu.VMEM((_LW, _K), jnp.float32),
        pltpu.VMEM((_LW, _K), jnp.float32),
        pltpu.VMEM((_LW, _BW), jnp.float32),
        pltpu.VMEM((_K,), jnp.float32),
        pltpu.VMEM((_K,), jnp.float32),
        pltpu.VMEM((_K,), jnp.float32),
        pltpu.VMEM((_K,), jnp.float32),
    ],
)(_sc_body)


def _tc_body(z_ref, mu_ref, lv_ref, w_ref, out_ref):
    z = z_ref[...]
    mu = mu_ref[...]
    lv = lv_ref[...]
    w = w_ref[...]
    wmax = jnp.max(w)
    lw = w - (wmax + jnp.log(jnp.sum(jnp.exp(w - wmax))))
    nhalfprec = -0.5 * jnp.exp(-lv)
    base = (-_C1 - 0.5 * lv) + lw[0][:, None]
    diff = z[None, :, :] - mu[:, None, :]
    t = base[:, None, :] + nhalfprec[:, None, :] * (diff * diff)
    m = jnp.max(t, axis=0)
    s = jnp.sum(jnp.exp(t - m[None, :, :]), axis=0)
    out_ref[...] = m + jnp.log(s)


def _tc_part(z2d_full, means, logvars, w2):
    return pl.pallas_call(
        _tc_body,
        grid=(_RTC // _BB,),
        in_specs=[
            pl.BlockSpec((_BB, _W), lambda i: (i, 0)),
            pl.BlockSpec((_K, _W), lambda i: (0, 0)),
            pl.BlockSpec((_K, _W), lambda i: (0, 0)),
            pl.BlockSpec((1, _K), lambda i: (0, 0)),
        ],
        out_specs=pl.BlockSpec((_BB, _W), lambda i: (i, 0)),
        out_shape=jax.ShapeDtypeStruct((_RTC, _W), jnp.float32),
    )(z2d_full, means, logvars, w2)


@jax.jit
def kernel(z, means, logvars, w):
    z2d_full = z.reshape(_B // 2, _W)
    z_sc = (z[_BTC:].T.reshape(_LG, _LW, _BG, _BW)
            .transpose(0, 2, 1, 3).reshape(_NW, _LW, _BW))
    mu2 = jnp.tile(means, (1, 2))
    lv2 = jnp.tile(logvars, (1, 2))
    w2 = w.reshape(1, _K)

    out_sc3 = _sc_kernel(z_sc, means.T, logvars.T, w.reshape(_K))
    out_tc = _tc_part(z2d_full, mu2, lv2, w2)

    out_tc_lb = out_tc.reshape(_BTC, _L).T
    out_sc_lb = (out_sc3.reshape(_LG, _BG, _LW, _BW)
                 .transpose(0, 2, 1, 3).reshape(_L, _BSC))
    return jnp.concatenate([out_tc_lb, out_sc_lb], axis=1)


# speedup vs baseline: 1.0620x; 1.0620x over previous
import functools
import math

import jax
import jax.numpy as jnp
from jax import lax
from jax.experimental import pallas as pl
from jax.experimental.pallas import tpu as pltpu
from jax.experimental.pallas import tpu_sc as plsc

_L = 64
_K = 128
_B = 4096
_NW = 32
_BTC = 3072
_BSC = _B - _BTC
_LG = 4
_BG = 8
_LW = _L // _LG
_BW = _BSC // _BG
_NC = _BW // 16
_KC = _K // 16
_C1 = 0.5 * math.log(2.0 * math.pi)
_LN2 = math.log(2.0)
_NEG_BIG = -3.0e38
_S_MIN = 1e-25

_W = 2 * _L
_RTC = _BTC // 2
_BB = 64


_GATHER_DNUMS = lax.GatherDimensionNumbers(
    offset_dims=(), collapsed_slice_dims=(0,), start_index_map=(0,)
)


def _shuf(v, idx):
    return lax.gather(
        v, idx.reshape(16, 1), _GATHER_DNUMS, (1,),
        mode=lax.GatherScatterMode.PROMISE_IN_BOUNDS,
    )


def _butterfly(v, op):
    lane = lax.iota(jnp.int32, 16)
    for b in range(4):
        v = op(v, _shuf(v, lane ^ (1 << b)))
    return v


def _log16(v):
    bits = lax.bitcast_convert_type(v, jnp.int32)
    e = (bits >> 23) - 127
    mbits = (bits & jnp.int32(0x007FFFFF)) | jnp.int32(0x3F800000)
    m = lax.bitcast_convert_type(mbits, jnp.float32)
    big = m > 1.4142135
    m = jnp.where(big, 0.5 * m, m)
    ef = e.astype(jnp.float32) + jnp.where(big, 1.0, 0.0)
    t = (m - 1.0) / (m + 1.0)
    u = t * t
    p = 2.0 * t * (1.0 + u * (1.0 / 3.0 + u * (0.2 + u * (1.0 / 7.0))))
    return ef * _LN2 + p


def _sc_body(zt, mut, lvt, wvec, out3, zv, muv, lvv, outv, av, cv, dv, wv):
    cid = lax.axis_index("c")
    sid = lax.axis_index("s")
    wid = cid * 16 + sid
    lg = wid >> 3
    pltpu.sync_copy(zt.at[wid], zv)
    pltpu.sync_copy(mut.at[pl.ds(lg * _LW, _LW)], muv)
    pltpu.sync_copy(lvt.at[pl.ds(lg * _LW, _LW)], lvv)
    pltpu.sync_copy(wvec, wv)

    wmax_acc = wv[pl.ds(0, 16)]
    for kc in range(1, _KC):
        wmax_acc = jnp.maximum(wmax_acc, wv[pl.ds(kc * 16, 16)])
    wm = _butterfly(wmax_acc, jnp.maximum)
    se = jnp.zeros((16,), jnp.float32)
    for kc in range(_KC):
        se = se + jnp.exp(wv[pl.ds(kc * 16, 16)] - wm)
    lse16 = wm + _log16(_butterfly(se, jnp.add))

    def l_body(l, carry):
        pkmax = jnp.full((16,), _NEG_BIG)
        for kc in range(_KC):
            sl = pl.ds(kc * 16, 16)
            lv16 = lvv[l, sl]
            mu16 = muv[l, sl]
            prec = jnp.exp(-lv16)
            a16 = -0.5 * prec
            c16 = prec * mu16
            pk16 = (-_C1 - 0.5 * lv16) + (wv[sl] - lse16)
            av[sl] = a16
            cv[sl] = c16
            dv[sl] = pk16 + a16 * (mu16 * mu16)
            pkmax = jnp.maximum(pkmax, pk16)
        m0 = _butterfly(pkmax, jnp.maximum)
        for kc in range(_KC):
            sl = pl.ds(kc * 16, 16)
            dv[sl] = dv[sl] - m0

        z8 = [zv[l, pl.ds(ci * 16, 16)] for ci in range(_NC)]
        z28 = [zz * zz for zz in z8]

        def k_body(kc, ss):
            a16 = av[pl.ds(kc * 16, 16)]
            c16 = cv[pl.ds(kc * 16, 16)]
            d16 = dv[pl.ds(kc * 16, 16)]
            for j in range(16):
                ak = a16[j]
                ck = c16[j]
                dk = d16[j]
                ss = tuple(
                    ss[ci] + jnp.exp((ak * z28[ci] + ck * z8[ci]) + dk)
                    for ci in range(_NC)
                )
            return ss

        s8 = lax.fori_loop(
            0, _KC, k_body,
            tuple(jnp.zeros((16,), jnp.float32) for _ in range(_NC)),
        )

        smin = s8[0]
        for ci in range(1, _NC):
            smin = jnp.minimum(smin, s8[ci])
        bad = _butterfly(smin, jnp.minimum)[0] < _S_MIN

        for ci in range(_NC):
            outv[l, pl.ds(ci * 16, 16)] = _log16(s8[ci]) + m0

        @pl.when(bad)
        def redo():
            def mx_body(kc, mm):
                a16 = av[pl.ds(kc * 16, 16)]
                c16 = cv[pl.ds(kc * 16, 16)]
                d16 = dv[pl.ds(kc * 16, 16)]
                for j in range(16):
                    ak = a16[j]
                    ck = c16[j]
                    dk = d16[j]
                    mm = tuple(
                        jnp.maximum(mm[ci], (ak * z28[ci] + ck * z8[ci]) + dk)
                        for ci in range(_NC)
                    )
                return mm

            m8 = lax.fori_loop(
                0, _KC, mx_body,
                tuple(jnp.full((16,), _NEG_BIG) for _ in range(_NC)),
            )

            def s2_body(kc, ss):
                a16 = av[pl.ds(kc * 16, 16)]
                c16 = cv[pl.ds(kc * 16, 16)]
                d16 = dv[pl.ds(kc * 16, 16)]
                for j in range(16):
                    ak = a16[j]
                    ck = c16[j]
                    dk = d16[j]
                    ss = tuple(
                        ss[ci] + jnp.exp(((ak * z28[ci] + ck * z8[ci]) + dk) - m8[ci])
                        for ci in range(_NC)
                    )
                return ss

            s28 = lax.fori_loop(
                0, _KC, s2_body,
                tuple(jnp.zeros((16,), jnp.float32) for _ in range(_NC)),
            )
            for ci in range(_NC):
                outv[l, pl.ds(ci * 16, 16)] = (m8[ci] + _log16(s28[ci])) + m0

        return carry

    lax.fori_loop(0, _LW, l_body, 0)
    pltpu.sync_copy(outv, out3.at[wid])


_sc_kernel = functools.partial(
    pl.kernel,
    mesh=plsc.VectorSubcoreMesh(core_axis_name="c", subcore_axis_name="s"),
    out_type=jax.ShapeDtypeStruct((_NW, _LW, _BW), jnp.float32),
    scratch_types=[
        pltpu.VMEM((_LW, _BW), jnp.float32),
        pltpu.VMEM((_LW, _K), jnp.float32),
        pltpu.VMEM((_LW, _K), jnp.float32),
        pltpu.VMEM((_LW, _BW), jnp.float32),
        pltpu.VMEM((_K,), jnp.float32),
        pltpu.VMEM((_K,), jnp.float32),
        pltpu.VMEM((_K,), jnp.float32),
        pltpu.VMEM((_K,), jnp.float32),
    ],
)(_sc_body)


def _tc_body(z_ref, mu_ref, lv_ref, w_ref, out_ref, av, cv, dv, m0v):
    @pl.when(pl.program_id(0) == 0)
    def init():
        mu = mu_ref[...]
        lv = lv_ref[...]
        w = w_ref[...]
        wmax = jnp.max(w)
        lw = w - (wmax + jnp.log(jnp.sum(jnp.exp(w - wmax))))
        prec = jnp.exp(-lv)
        a = -0.5 * prec
        pk = (-_C1 - 0.5 * lv) + lw[0][:, None]
        m0 = jnp.max(pk, axis=0, keepdims=True)
        av[...] = a
        cv[...] = prec * mu
        dv[...] = (pk + a * (mu * mu)) - m0
        m0v[...] = m0

    z = z_ref[...]
    z2 = z * z

    def k_body(k, s):
        return s + jnp.exp((av[k] * z2 + cv[k] * z) + dv[k])

    s = lax.fori_loop(0, _K, k_body, jnp.zeros((_BB, _W), jnp.float32))
    m0 = m0v[...]
    out_ref[...] = jnp.log(s) + m0

    @pl.when(jnp.any(s < _S_MIN))
    def redo():
        def mx_body(k, m):
            return jnp.maximum(m, (av[k] * z2 + cv[k] * z) + dv[k])

        m = lax.fori_loop(0, _K, mx_body, jnp.full((_BB, _W), _NEG_BIG))

        def s2_body(k, s2):
            return s2 + jnp.exp(((av[k] * z2 + cv[k] * z) + dv[k]) - m)

        s2 = lax.fori_loop(0, _K, s2_body, jnp.zeros((_BB, _W), jnp.float32))
        out_ref[...] = (m + jnp.log(s2)) + m0


def _tc_part(z2d_full, means, logvars, w2):
    return pl.pallas_call(
        _tc_body,
        grid=(_RTC // _BB,),
        in_specs=[
            pl.BlockSpec((_BB, _W), lambda i: (i, 0)),
            pl.BlockSpec((_K, _W), lambda i: (0, 0)),
            pl.BlockSpec((_K, _W), lambda i: (0, 0)),
            pl.BlockSpec((1, _K), lambda i: (0, 0)),
        ],
        out_specs=pl.BlockSpec((_BB, _W), lambda i: (i, 0)),
        out_shape=jax.ShapeDtypeStruct((_RTC, _W), jnp.float32),
        scratch_shapes=[
            pltpu.VMEM((_K, _W), jnp.float32),
            pltpu.VMEM((_K, _W), jnp.float32),
            pltpu.VMEM((_K, _W), jnp.float32),
            pltpu.VMEM((1, _W), jnp.float32),
        ],
    )(z2d_full, means, logvars, w2)


@jax.jit
def kernel(z, means, logvars, w):
    z2d_full = z.reshape(_B // 2, _W)
    z_sc = (z[_BTC:].T.reshape(_LG, _LW, _BG, _BW)
            .transpose(0, 2, 1, 3).reshape(_NW, _LW, _BW))
    mu2 = jnp.tile(means, (1, 2))
    lv2 = jnp.tile(logvars, (1, 2))
    w2 = w.reshape(1, _K)

    out_sc3 = _sc_kernel(z_sc, means.T, logvars.T, w.reshape(_K))
    out_tc = _tc_part(z2d_full, mu2, lv2, w2)

    out_tc_lb = out_tc.reshape(_BTC, _L).T
    out_sc_lb = (out_sc3.reshape(_LG, _BG, _LW, _BW)
                 .transpose(0, 2, 1, 3).reshape(_L, _BSC))
    return jnp.concatenate([out_tc_lb, out_sc_lb], axis=1)


# speedup vs baseline: 1.4904x; 1.4035x over previous
import functools
import math

import jax
import jax.numpy as jnp
from jax import lax
from jax.experimental import pallas as pl
from jax.experimental.pallas import tpu as pltpu
from jax.experimental.pallas import tpu_sc as plsc

_L = 64
_K = 128
_B = 4096
_NW = 32
_BTC = 3072
_BSC = _B - _BTC
_LG = 4
_BG = 8
_LW = _L // _LG
_BW = _BSC // _BG
_NC = _BW // 16
_KC = _K // 16
_C1 = 0.5 * math.log(2.0 * math.pi)
_LN2 = math.log(2.0)
_NEG_BIG = -3.0e38
_S_MIN = 1e-25

_W = 2 * _L
_RTC = _BTC // 2
_BB = 64


_GATHER_DNUMS = lax.GatherDimensionNumbers(
    offset_dims=(), collapsed_slice_dims=(0,), start_index_map=(0,)
)


def _shuf(v, idx):
    return lax.gather(
        v, idx.reshape(16, 1), _GATHER_DNUMS, (1,),
        mode=lax.GatherScatterMode.PROMISE_IN_BOUNDS,
    )


def _butterfly(v, op):
    lane = lax.iota(jnp.int32, 16)
    for b in range(4):
        v = op(v, _shuf(v, lane ^ (1 << b)))
    return v


def _log16(v):
    bits = lax.bitcast_convert_type(v, jnp.int32)
    e = (bits >> 23) - 127
    mbits = (bits & jnp.int32(0x007FFFFF)) | jnp.int32(0x3F800000)
    m = lax.bitcast_convert_type(mbits, jnp.float32)
    big = m > 1.4142135
    m = jnp.where(big, 0.5 * m, m)
    ef = e.astype(jnp.float32) + jnp.where(big, 1.0, 0.0)
    t = (m - 1.0) / (m + 1.0)
    u = t * t
    p = 2.0 * t * (1.0 + u * (1.0 / 3.0 + u * (0.2 + u * (1.0 / 7.0))))
    return ef * _LN2 + p


def _sc_body(zt, mut, lvt, wvec, out3, zv, muv, lvv, outv, av, cv, dv, wv):
    cid = lax.axis_index("c")
    sid = lax.axis_index("s")
    wid = cid * 16 + sid
    lg = wid >> 3
    pltpu.sync_copy(zt.at[wid], zv)
    pltpu.sync_copy(mut.at[pl.ds(lg * _LW, _LW)], muv)
    pltpu.sync_copy(lvt.at[pl.ds(lg * _LW, _LW)], lvv)
    pltpu.sync_copy(wvec, wv)

    wmax_acc = wv[pl.ds(0, 16)]
    for kc in range(1, _KC):
        wmax_acc = jnp.maximum(wmax_acc, wv[pl.ds(kc * 16, 16)])
    wm = _butterfly(wmax_acc, jnp.maximum)
    se = jnp.zeros((16,), jnp.float32)
    for kc in range(_KC):
        se = se + jnp.exp(wv[pl.ds(kc * 16, 16)] - wm)
    lse16 = wm + _log16(_butterfly(se, jnp.add))

    def l_body(l, carry):
        pkmax = jnp.full((16,), _NEG_BIG)
        for kc in range(_KC):
            sl = pl.ds(kc * 16, 16)
            lv16 = lvv[l, sl]
            mu16 = muv[l, sl]
            prec = jnp.exp(-lv16)
            a16 = -0.5 * prec
            c16 = prec * mu16
            pk16 = (-_C1 - 0.5 * lv16) + (wv[sl] - lse16)
            av[sl] = a16
            cv[sl] = c16
            dv[sl] = pk16 + a16 * (mu16 * mu16)
            pkmax = jnp.maximum(pkmax, pk16)
        m0 = _butterfly(pkmax, jnp.maximum)
        for kc in range(_KC):
            sl = pl.ds(kc * 16, 16)
            dv[sl] = dv[sl] - m0

        z8 = [zv[l, pl.ds(ci * 16, 16)] for ci in range(_NC)]
        z28 = [zz * zz for zz in z8]

        def k_body(kc, ss):
            a16 = av[pl.ds(kc * 16, 16)]
            c16 = cv[pl.ds(kc * 16, 16)]
            d16 = dv[pl.ds(kc * 16, 16)]
            for j in range(16):
                ak = a16[j]
                ck = c16[j]
                dk = d16[j]
                ss = tuple(
                    ss[ci] + jnp.exp((ak * z28[ci] + ck * z8[ci]) + dk)
                    for ci in range(_NC)
                )
            return ss

        s8 = lax.fori_loop(
            0, _KC, k_body,
            tuple(jnp.zeros((16,), jnp.float32) for _ in range(_NC)),
        )

        smin = s8[0]
        for ci in range(1, _NC):
            smin = jnp.minimum(smin, s8[ci])
        bad = _butterfly(smin, jnp.minimum)[0] < _S_MIN

        for ci in range(_NC):
            outv[l, pl.ds(ci * 16, 16)] = _log16(s8[ci]) + m0

        @pl.when(bad)
        def redo():
            def mx_body(kc, mm):
                a16 = av[pl.ds(kc * 16, 16)]
                c16 = cv[pl.ds(kc * 16, 16)]
                d16 = dv[pl.ds(kc * 16, 16)]
                for j in range(16):
                    ak = a16[j]
                    ck = c16[j]
                    dk = d16[j]
                    mm = tuple(
                        jnp.maximum(mm[ci], (ak * z28[ci] + ck * z8[ci]) + dk)
                        for ci in range(_NC)
                    )
                return mm

            m8 = lax.fori_loop(
                0, _KC, mx_body,
                tuple(jnp.full((16,), _NEG_BIG) for _ in range(_NC)),
            )

            def s2_body(kc, ss):
                a16 = av[pl.ds(kc * 16, 16)]
                c16 = cv[pl.ds(kc * 16, 16)]
                d16 = dv[pl.ds(kc * 16, 16)]
                for j in range(16):
                    ak = a16[j]
                    ck = c16[j]
                    dk = d16[j]
                    ss = tuple(
                        ss[ci] + jnp.exp(((ak * z28[ci] + ck * z8[ci]) + dk) - m8[ci])
                        for ci in range(_NC)
                    )
                return ss

            s28 = lax.fori_loop(
                0, _KC, s2_body,
                tuple(jnp.zeros((16,), jnp.float32) for _ in range(_NC)),
            )
            for ci in range(_NC):
                outv[l, pl.ds(ci * 16, 16)] = (m8[ci] + _log16(s28[ci])) + m0

        return carry

    lax.fori_loop(0, _LW, l_body, 0)
    pltpu.sync_copy(outv, out3.at[wid])


_sc_kernel = functools.partial(
    pl.kernel,
    mesh=plsc.VectorSubcoreMesh(core_axis_name="c", subcore_axis_name="s"),
    out_type=jax.ShapeDtypeStruct((_NW, _LW, _BW), jnp.float32),
    scratch_types=[
        pltpu.VMEM((_LW, _BW), jnp.float32),
        pltpu.VMEM((_LW, _K), jnp.float32),
        pltpu.VMEM((_LW, _K), jnp.float32),
        pltpu.VMEM((_LW, _BW), jnp.float32),
        pltpu.VMEM((_K,), jnp.float32),
        pltpu.VMEM((_K,), jnp.float32),
        pltpu.VMEM((_K,), jnp.float32),
        pltpu.VMEM((_K,), jnp.float32),
    ],
)(_sc_body)


def _tc_body(z_ref, mu_ref, lv_ref, w_ref, out_ref, av, cv, dv, m0v):
    @pl.when(pl.program_id(0) == 0)
    def init():
        mu = mu_ref[...]
        lv = lv_ref[...]
        w = w_ref[...]
        wmax = jnp.max(w)
        lw = w - (wmax + jnp.log(jnp.sum(jnp.exp(w - wmax))))
        prec = jnp.exp(-lv)
        a = -0.5 * prec
        pk = (-_C1 - 0.5 * lv) + lw[0][:, None]
        m0 = jnp.max(pk, axis=0, keepdims=True)
        av[...] = a
        cv[...] = prec * mu
        dv[...] = (pk + a * (mu * mu)) - m0
        m0v[...] = m0

    z = z_ref[...]
    z2 = z * z

    def k_body(k, s):
        return s + jnp.exp((av[k] * z2 + cv[k] * z) + dv[k])

    s = lax.fori_loop(0, _K, k_body, jnp.zeros((_BB, _W), jnp.float32),
                      unroll=8)
    m0 = m0v[...]
    out_ref[...] = jnp.log(s) + m0

    @pl.when(jnp.any(s < _S_MIN))
    def redo():
        def mx_body(k, m):
            return jnp.maximum(m, (av[k] * z2 + cv[k] * z) + dv[k])

        m = lax.fori_loop(0, _K, mx_body, jnp.full((_BB, _W), _NEG_BIG))

        def s2_body(k, s2):
            return s2 + jnp.exp(((av[k] * z2 + cv[k] * z) + dv[k]) - m)

        s2 = lax.fori_loop(0, _K, s2_body, jnp.zeros((_BB, _W), jnp.float32))
        out_ref[...] = (m + jnp.log(s2)) + m0


def _tc_part(z2d_full, means, logvars, w2):
    return pl.pallas_call(
        _tc_body,
        grid=(_RTC // _BB,),
        in_specs=[
            pl.BlockSpec((_BB, _W), lambda i: (i, 0)),
            pl.BlockSpec((_K, _W), lambda i: (0, 0)),
            pl.BlockSpec((_K, _W), lambda i: (0, 0)),
            pl.BlockSpec((1, _K), lambda i: (0, 0)),
        ],
        out_specs=pl.BlockSpec((_BB, _W), lambda i: (i, 0)),
        out_shape=jax.ShapeDtypeStruct((_RTC, _W), jnp.float32),
        scratch_shapes=[
            pltpu.VMEM((_K, _W), jnp.float32),
            pltpu.VMEM((_K, _W), jnp.float32),
            pltpu.VMEM((_K, _W), jnp.float32),
            pltpu.VMEM((1, _W), jnp.float32),
        ],
    )(z2d_full, means, logvars, w2)


@jax.jit
def kernel(z, means, logvars, w):
    z2d_full = z.reshape(_B // 2, _W)
    z_sc = (z[_BTC:].T.reshape(_LG, _LW, _BG, _BW)
            .transpose(0, 2, 1, 3).reshape(_NW, _LW, _BW))
    mu2 = jnp.tile(means, (1, 2))
    lv2 = jnp.tile(logvars, (1, 2))
    w2 = w.reshape(1, _K)

    out_sc3 = _sc_kernel(z_sc, means.T, logvars.T, w.reshape(_K))
    out_tc = _tc_part(z2d_full, mu2, lv2, w2)

    out_tc_lb = out_tc.reshape(_BTC, _L).T
    out_sc_lb = (out_sc3.reshape(_LG, _BG, _LW, _BW)
                 .transpose(0, 2, 1, 3).reshape(_L, _BSC))
    return jnp.concatenate([out_tc_lb, out_sc_lb], axis=1)
